# baseline (speedup 1.0000x reference)
import functools

import jax
import jax.numpy as jnp
from jax import lax
from jax.experimental import pallas as pl
from jax.experimental.pallas import tpu as pltpu
from jax.experimental.pallas import tpu_sc as plsc

D = 768
T = 77
B = 256

NC = 2
NS = 16
NW = NC * NS
ROWS_PER_W = B * T // NW
CHUNK = 56
NCHUNK = ROWS_PER_W // CHUNK
LANES = 16
GROUPS = D // LANES


def _make_kernel():
    mesh = plsc.VectorSubcoreMesh(core_axis_name="c", subcore_axis_name="s")

    @functools.partial(
        pl.kernel,
        mesh=mesh,
        out_type=jax.ShapeDtypeStruct((B * T, D), jnp.float32),
        scratch_types=[
            pltpu.VMEM((ROWS_PER_W,), jnp.int32),
            pltpu.VMEM((T, D), jnp.float32),
            pltpu.VMEM((CHUNK, D), jnp.float32),
            pltpu.SemaphoreType.DMA,
        ],
    )
    def k(tokens_hbm, table_hbm, pos_hbm, out_hbm, idx_v, pos_v, rows_v, sem):
        wid = lax.axis_index("s") * NC + lax.axis_index("c")
        row0 = wid * ROWS_PER_W
        pltpu.sync_copy(tokens_hbm.at[pl.ds(row0, ROWS_PER_W)], idx_v)
        pltpu.sync_copy(pos_hbm, pos_v)
        for c in range(NCHUNK):
            pltpu.async_copy(
                table_hbm.at[idx_v.at[pl.ds(c * CHUNK, CHUNK)]], rows_v, sem
            ).wait()
            pbase = (c * CHUNK) % T

            def body(r, _):
                p = lax.rem(pbase + r, T)
                for g in range(GROUPS):
                    sl = pl.ds(g * LANES, LANES)
                    rows_v[r, sl] = rows_v[r, sl] + pos_v[p, sl]
                return 0

            lax.fori_loop(0, CHUNK, body, 0)
            pltpu.sync_copy(
                rows_v, out_hbm.at[pl.ds(row0 + c * CHUNK, CHUNK)]
            )

    return k


_grid_kernel = _make_kernel()


def kernel(tokens, token_embedding, position_embedding):
    out = _grid_kernel(tokens.reshape(-1).astype(jnp.int32), token_embedding,
                       position_embedding)
    return out.reshape(B, T, D)


# speedup vs baseline: 1.2186x; 1.2186x over previous
import functools

import jax
import jax.numpy as jnp
from jax import lax
from jax.experimental import pallas as pl
from jax.experimental.pallas import tpu as pltpu
from jax.experimental.pallas import tpu_sc as plsc

D = 768
T = 77
B = 256

NC = 2
NS = 16
NW = NC * NS
ROWS_PER_W = B * T // NW
CHUNK = 40
LANES = 16
GROUPS = D // LANES

_CHUNKS = [(i * CHUNK, CHUNK) for i in range(ROWS_PER_W // CHUNK)]
if ROWS_PER_W % CHUNK:
    _CHUNKS.append((ROWS_PER_W - ROWS_PER_W % CHUNK, ROWS_PER_W % CHUNK))


def _make_kernel():
    mesh = plsc.VectorSubcoreMesh(core_axis_name="c", subcore_axis_name="s")

    @functools.partial(
        pl.kernel,
        mesh=mesh,
        out_type=jax.ShapeDtypeStruct((B * T, D), jnp.float32),
        scratch_types=[
            pltpu.VMEM((ROWS_PER_W,), jnp.int32),
            pltpu.VMEM((T, D), jnp.float32),
            pltpu.VMEM((CHUNK, D), jnp.float32),
            pltpu.VMEM((CHUNK, D), jnp.float32),
            pltpu.SemaphoreType.DMA,
            pltpu.SemaphoreType.DMA,
            pltpu.SemaphoreType.DMA,
            pltpu.SemaphoreType.DMA,
        ],
    )
    def k(tokens_hbm, table_hbm, pos_hbm, out_hbm,
          idx_v, pos_v, rows_a, rows_b, sg0, sg1, sw0, sw1):
        wid = lax.axis_index("s") * NC + lax.axis_index("c")
        row0 = wid * ROWS_PER_W
        pltpu.sync_copy(tokens_hbm.at[pl.ds(row0, ROWS_PER_W)], idx_v)
        pltpu.sync_copy(pos_hbm, pos_v)

        bufs = (rows_a, rows_b)
        sems_g = (sg0, sg1)
        sems_w = (sw0, sw1)
        n = len(_CHUNKS)
        gathers = {}
        writes = {}

        def issue_gather(ci):
            start, size = _CHUNKS[ci]
            buf = bufs[ci % 2]
            gathers[ci] = pltpu.async_copy(
                table_hbm.at[idx_v.at[pl.ds(start, size)]],
                buf.at[pl.ds(0, size)],
                sems_g[ci % 2],
            )

        issue_gather(0)
        for ci, (start, size) in enumerate(_CHUNKS):
            buf = bufs[ci % 2]
            gathers[ci].wait()
            if ci >= 1:
                writes[ci - 1].wait()
            if ci + 1 < n:
                issue_gather(ci + 1)

            def body(r, _):
                p = lax.rem(start + r, T)
                for g in range(GROUPS):
                    sl = pl.ds(g * LANES, LANES)
                    plsc.addupdate(buf.at[r, sl], pos_v[p, sl])
                return 0

            lax.fori_loop(0, size, body, 0)
            writes[ci] = pltpu.async_copy(
                buf.at[pl.ds(0, size)],
                out_hbm.at[pl.ds(row0 + start, size)],
                sems_w[ci % 2],
            )
        writes[n - 1].wait()

    return k


_grid_kernel = _make_kernel()


def kernel(tokens, token_embedding, position_embedding):
    out = _grid_kernel(tokens.reshape(-1).astype(jnp.int32), token_embedding,
                       position_embedding)
    return out.reshape(B, T, D)


# speedup vs baseline: 1.5904x; 1.3051x over previous
import functools

import jax
import jax.numpy as jnp
from jax import lax
from jax.experimental import pallas as pl
from jax.experimental.pallas import tpu as pltpu
from jax.experimental.pallas import tpu_sc as plsc

D = 768
T = 77
TPAD = 80
B = 256

NC = 2
NS = 16
NW = NC * NS
SEQ_PER_W = B // NW
CHUNK0 = 40
CHUNK1 = T - CHUNK0
LANES = 16
GROUPS = D // LANES

_TCHUNKS = [(0, CHUNK0), (CHUNK0, CHUNK1)]


def _make_kernel():
    mesh = plsc.VectorSubcoreMesh(core_axis_name="c", subcore_axis_name="s")

    @functools.partial(
        pl.kernel,
        mesh=mesh,
        out_type=jax.ShapeDtypeStruct((B, T, D), jnp.float32),
        scratch_types=[
            pltpu.VMEM((SEQ_PER_W * TPAD,), jnp.int32),
            pltpu.VMEM((T, D), jnp.float32),
            pltpu.VMEM((CHUNK0, D), jnp.float32),
            pltpu.VMEM((CHUNK1, D), jnp.float32),
            pltpu.SemaphoreType.DMA,
            pltpu.SemaphoreType.DMA,
            pltpu.SemaphoreType.DMA,
            pltpu.SemaphoreType.DMA,
        ],
    )
    def k(tokens_hbm, table_hbm, pos_hbm, out_hbm,
          idx_v, pos_v, rows_a, rows_b, sg0, sg1, sw0, sw1):
        wid = lax.axis_index("s") * NC + lax.axis_index("c")
        seq0 = wid * SEQ_PER_W
        pltpu.sync_copy(tokens_hbm.at[pl.ds(seq0 * TPAD, SEQ_PER_W * TPAD)],
                        idx_v)
        pltpu.sync_copy(pos_hbm, pos_v)

        bufs = (rows_a, rows_b)
        sems_g = (sg0, sg1)
        sems_w = (sw0, sw1)
        chunks = [(s, t0, sz) for s in range(SEQ_PER_W) for t0, sz in _TCHUNKS]
        n = len(chunks)
        gathers = {}
        writes = {}

        def issue_gather(ci):
            s, t0, sz = chunks[ci]
            gathers[ci] = pltpu.async_copy(
                table_hbm.at[idx_v.at[pl.ds(s * TPAD + t0, sz)]],
                bufs[ci % 2],
                sems_g[ci % 2],
            )

        issue_gather(0)
        for ci, (s, t0, sz) in enumerate(chunks):
            buf = bufs[ci % 2]
            gathers[ci].wait()
            if ci >= 1:
                writes[ci - 1].wait()
            if ci + 1 < n:
                issue_gather(ci + 1)

            def body(r, _):
                for g in range(GROUPS):
                    sl = pl.ds(g * LANES, LANES)
                    plsc.addupdate(buf.at[r, sl], pos_v[t0 + r, sl])
                return 0

            lax.fori_loop(0, sz, body, 0)
            writes[ci] = pltpu.async_copy(
                buf,
                out_hbm.at[seq0 + s, pl.ds(t0, sz)],
                sems_w[ci % 2],
            )
        writes[n - 1].wait()

    return k


_grid_kernel = _make_kernel()


def kernel(tokens, token_embedding, position_embedding):
    tok = jnp.pad(tokens.astype(jnp.int32), ((0, 0), (0, TPAD - T)))
    return _grid_kernel(tok.reshape(-1), token_embedding, position_embedding)
